# Initial kernel scaffold; baseline (speedup 1.0000x reference)
import jax
import jax.numpy as jnp
from jax.experimental import pallas as pl


def kernel(x, table):
    raise NotImplementedError("write your pallas kernel here")


# speedup vs baseline: 3.8547x; 3.8547x over previous
import functools
import math

import jax
import jax.numpy as jnp
from jax import lax
from jax.experimental import pallas as pl
from jax.experimental.pallas import tpu as pltpu
from jax.experimental.pallas import tpu_sc as plsc

SEQ = 200
D = 64
BATCH = 4096
B = BATCH * SEQ
NC = 2
NS = 16
NW = NC * NS
BPW = B // NW
GB = 100
CHUNK = 1600
NGATH = CHUNK // GB
NCHUNK = BPW // CHUNK
XROWS = B // GB


def _pe_body(out_ref):
    pos = lax.broadcasted_iota(jnp.int32, (SEQ, D), 0).astype(jnp.float32)
    col = lax.broadcasted_iota(jnp.int32, (SEQ, D), 1)
    k = (col // 2) * 2
    angle = pos * jnp.exp(k.astype(jnp.float32) * (-math.log(10000.0) / D))
    out_ref[...] = jnp.where(col % 2 == 0, jnp.sin(angle), jnp.cos(angle))


_pe_table = pl.pallas_call(
    _pe_body, out_shape=jax.ShapeDtypeStruct((SEQ, D), jnp.float32))


@functools.partial(
    pl.kernel,
    out_type=jax.ShapeDtypeStruct((B, D), jnp.float32),
    mesh=plsc.VectorSubcoreMesh(core_axis_name="c", subcore_axis_name="s"),
    scratch_types=[
        pltpu.VMEM((NGATH, GB), jnp.int32),
        pltpu.VMEM((CHUNK, D), jnp.float32),
        pltpu.VMEM((SEQ, D), jnp.float32),
        pltpu.SemaphoreType.DMA,
    ],
    compiler_params=pltpu.CompilerParams(use_tc_tiling_on_sc=False),
)
def _sc_lookup(table_hbm, x_hbm, pe_hbm, out_hbm, idx_v, rows_v, pe_v, sem):
    wid = lax.axis_index("s") * NC + lax.axis_index("c")
    pltpu.sync_copy(pe_hbm, pe_v)

    @pl.loop(0, NCHUNK)
    def _chunk(c):
        rowbase = pl.multiple_of(wid * (BPW // GB) + c * NGATH, NGATH)
        base = rowbase * GB
        pltpu.sync_copy(x_hbm.at[pl.ds(rowbase, NGATH)], idx_v)
        copies = [
            pltpu.async_copy(
                table_hbm.at[idx_v.at[j]],
                rows_v.at[pl.ds(j * GB, GB)],
                sem,
            )
            for j in range(NGATH)
        ]
        for cp in copies:
            cp.wait()

        @pl.loop(0, SEQ)
        def _add(p):
            for g in range(D // 16):
                sl = pl.ds(g * 16, 16)
                pev = pe_v[p, sl]
                for s in range(CHUNK // SEQ):
                    r = s * SEQ + p
                    rows_v[r, sl] = rows_v[r, sl] + pev

        pltpu.sync_copy(rows_v, out_hbm.at[pl.ds(base, CHUNK)])


def kernel(x, table):
    pe = _pe_table()
    x2d = x.reshape(XROWS, GB).astype(jnp.int32)
    out = _sc_lookup(table, x2d, pe)
    return out.reshape(x.shape + (D,))


# speedup vs baseline: 4.2344x; 1.0985x over previous
import functools
import math

import jax
import jax.numpy as jnp
from jax import lax
from jax.experimental import pallas as pl
from jax.experimental.pallas import tpu as pltpu
from jax.experimental.pallas import tpu_sc as plsc

SEQ = 200
D = 64
BATCH = 4096
B = BATCH * SEQ
NC = 2
NS = 16
NW = NC * NS
BPW = B // NW
GB = 100
SPC = 2
CHUNK = SPC * SEQ
NGATH = CHUNK // GB
NCHUNK = BPW // CHUNK
XROWS = B // GB
NBUF = 4


def _pe_body(out_ref):
    pos = lax.broadcasted_iota(jnp.int32, (SEQ, D), 0).astype(jnp.float32)
    col = lax.broadcasted_iota(jnp.int32, (SEQ, D), 1)
    k = (col // 2) * 2
    angle = pos * jnp.exp(k.astype(jnp.float32) * (-math.log(10000.0) / D))
    out_ref[...] = jnp.where(col % 2 == 0, jnp.sin(angle), jnp.cos(angle))


_pe_table = pl.pallas_call(
    _pe_body, out_shape=jax.ShapeDtypeStruct((SEQ, D), jnp.float32))


@functools.partial(
    pl.kernel,
    out_type=jax.ShapeDtypeStruct((BATCH, SEQ, D), jnp.float32),
    mesh=plsc.VectorSubcoreMesh(core_axis_name="c", subcore_axis_name="s"),
    scratch_types=(
        [pltpu.VMEM((NGATH, GB), jnp.int32) for _ in range(NBUF)]
        + [pltpu.VMEM((SPC, SEQ, D), jnp.float32) for _ in range(NBUF)]
        + [pltpu.VMEM((SEQ, D), jnp.float32)]
        + [pltpu.SemaphoreType.DMA] * (3 * NBUF)
    ),
    compiler_params=pltpu.CompilerParams(use_tc_tiling_on_sc=False),
)
def _sc_lookup(table_hbm, x_hbm, pe_hbm, out_hbm, *scratch):
    idxs = scratch[0:NBUF]
    rows = scratch[NBUF:2 * NBUF]
    pe_v = scratch[2 * NBUF]
    isems = scratch[2 * NBUF + 1:2 * NBUF + 1 + NBUF]
    gsems = scratch[2 * NBUF + 1 + NBUF:2 * NBUF + 1 + 2 * NBUF]
    ssems = scratch[2 * NBUF + 1 + 2 * NBUF:2 * NBUF + 1 + 3 * NBUF]

    wid = lax.axis_index("s") * NC + lax.axis_index("c")
    pltpu.sync_copy(pe_hbm, pe_v)

    def stage_idx(c, b):
        rowbase = pl.multiple_of(wid * (BPW // GB) + c * NGATH, NGATH)
        pltpu.async_copy(x_hbm.at[pl.ds(rowbase, NGATH)], idxs[b], isems[b])

    def wait_idx(b):
        pltpu.make_async_copy(
            x_hbm.at[pl.ds(0, NGATH)], idxs[b], isems[b]).wait()

    def fire_gathers(b):
        for j in range(NGATH):
            pltpu.async_copy(
                table_hbm.at[idxs[b].at[j]],
                rows[b].at[j // SPC, pl.ds((j % SPC) * GB, GB)],
                gsems[b],
            )

    def wait_gathers(b):
        for _ in range(NGATH):
            pltpu.make_async_copy(
                table_hbm.at[pl.ds(0, GB)],
                rows[b].at[0, pl.ds(0, GB)],
                gsems[b],
            ).wait()

    def add_pe(b):
        @pl.loop(0, SEQ // 2)
        def _add(q):
            for pp in range(2):
                p = 2 * q + pp
                for g in range(D // 16):
                    sl = pl.ds(g * 16, 16)
                    pev = pe_v[p, sl]
                    for s in range(SPC):
                        rows[b][s, p, sl] = rows[b][s, p, sl] + pev

    def fire_scatter(c, b):
        seqbase = pl.multiple_of(wid * (BPW // SEQ) + c * SPC, SPC)
        pltpu.async_copy(rows[b], out_hbm.at[pl.ds(seqbase, SPC)], ssems[b])

    def wait_scatter(b):
        pltpu.make_async_copy(
            rows[b], out_hbm.at[pl.ds(0, SPC)], ssems[b]).wait()

    def body(cc, b, fire, stage, skip_scatter_wait=False):
        gb = (b + 2) % NBUF
        if fire:
            if not skip_scatter_wait:
                wait_scatter(gb)
            wait_idx(gb)
            fire_gathers(gb)
        wait_gathers(b)
        if stage:
            stage_idx(cc + NBUF, b)
        add_pe(b)
        fire_scatter(cc, b)

    for c in range(NBUF):
        stage_idx(jnp.int32(c), c)
    wait_idx(0)
    fire_gathers(0)
    wait_idx(1)
    fire_gathers(1)
    body(jnp.int32(0), 0, fire=True, stage=True, skip_scatter_wait=True)
    body(jnp.int32(1), 1, fire=True, stage=True, skip_scatter_wait=True)

    @pl.loop(0, (NCHUNK - 8) // NBUF)
    def _grp(m):
        cc0 = 2 + NBUF * m
        for off in range(NBUF):
            body(cc0 + off, (2 + off) % NBUF, fire=True, stage=True)

    body(jnp.int32(NCHUNK - 6), 2, fire=True, stage=True)
    body(jnp.int32(NCHUNK - 5), 3, fire=True, stage=True)
    body(jnp.int32(NCHUNK - 4), 0, fire=True, stage=False)
    body(jnp.int32(NCHUNK - 3), 1, fire=True, stage=False)
    body(jnp.int32(NCHUNK - 2), 2, fire=False, stage=False)
    body(jnp.int32(NCHUNK - 1), 3, fire=False, stage=False)
    for b in range(NBUF):
        wait_scatter(b)


def kernel(x, table):
    pe = _pe_table()
    x2d = x.reshape(XROWS, GB).astype(jnp.int32)
    return _sc_lookup(table, x2d, pe)


# speedup vs baseline: 8.1095x; 1.9151x over previous
import functools
import math

import jax
import jax.numpy as jnp
from jax import lax
from jax.experimental import pallas as pl
from jax.experimental.pallas import tpu as pltpu
from jax.experimental.pallas import tpu_sc as plsc

SEQ = 200
D = 64
BATCH = 4096
B = BATCH * SEQ
NC = 2
NS = 16
NW = NC * NS
BPW = B // NW
GB = 100
SPC = 2
CHUNK = SPC * SEQ
NGATH = CHUNK // GB
NCHUNK = BPW // CHUNK
XROWS = B // GB
NBUF = 4
BN = 128
TPP = SEQ // 2


def _pe_body(out_ref):
    pos = lax.broadcasted_iota(jnp.int32, (SEQ, D), 0).astype(jnp.float32)
    col = lax.broadcasted_iota(jnp.int32, (SEQ, D), 1)
    k = (col // 2) * 2
    angle = pos * jnp.exp(k.astype(jnp.float32) * (-math.log(10000.0) / D))
    out_ref[...] = jnp.where(col % 2 == 0, jnp.sin(angle), jnp.cos(angle))


_pe_table = pl.pallas_call(
    _pe_body, out_shape=jax.ShapeDtypeStruct((SEQ, D), jnp.float32))


def _tr_body(g_ref, pe_ref, out_ref):
    pe = pe_ref[...]
    g3 = g_ref[...].reshape(BN, TPP, 128)
    for tp in range(TPP):
        blk = g3[:, tp, :]
        blkt = blk.T
        out_ref[2 * tp, :, :] = blkt[:D, :] + pe[2 * tp, :, None]
        out_ref[2 * tp + 1, :, :] = blkt[D:, :] + pe[2 * tp + 1, :, None]


_transpose_pe = pl.pallas_call(
    _tr_body,
    grid=(BATCH // BN,),
    in_specs=[
        pl.BlockSpec((BN * TPP, 128), lambda j: (j, 0)),
        pl.BlockSpec((SEQ, D), lambda j: (0, 0)),
    ],
    out_specs=pl.BlockSpec((SEQ, D, BN), lambda j: (0, 0, j)),
    out_shape=jax.ShapeDtypeStruct((SEQ, D, BATCH), jnp.float32),
)


@functools.partial(
    pl.kernel,
    out_type=jax.ShapeDtypeStruct((BATCH, SEQ, D), jnp.float32),
    mesh=plsc.VectorSubcoreMesh(core_axis_name="c", subcore_axis_name="s"),
    scratch_types=(
        [pltpu.VMEM((NGATH, GB), jnp.int32) for _ in range(NBUF)]
        + [pltpu.VMEM((SPC, SEQ, D), jnp.float32) for _ in range(NBUF)]
        + [pltpu.SemaphoreType.DMA] * (3 * NBUF)
    ),
    compiler_params=pltpu.CompilerParams(use_tc_tiling_on_sc=False),
)
def _sc_lookup(table_hbm, x_hbm, out_hbm, *scratch):
    idxs = scratch[0:NBUF]
    rows = scratch[NBUF:2 * NBUF]
    isems = scratch[2 * NBUF:3 * NBUF]
    gsems = scratch[3 * NBUF:4 * NBUF]
    ssems = scratch[4 * NBUF:5 * NBUF]

    wid = lax.axis_index("s") * NC + lax.axis_index("c")

    def stage_idx(c, b):
        rowbase = pl.multiple_of(wid * (BPW // GB) + c * NGATH, NGATH)
        pltpu.async_copy(x_hbm.at[pl.ds(rowbase, NGATH)], idxs[b], isems[b])

    def wait_idx(b):
        pltpu.make_async_copy(
            x_hbm.at[pl.ds(0, NGATH)], idxs[b], isems[b]).wait()

    def fire_gathers(b):
        for j in range(NGATH):
            pltpu.async_copy(
                table_hbm.at[idxs[b].at[j]],
                rows[b].at[j // SPC, pl.ds((j % SPC) * GB, GB)],
                gsems[b],
            )

    def wait_gathers(b):
        for _ in range(NGATH):
            pltpu.make_async_copy(
                table_hbm.at[pl.ds(0, GB)],
                rows[b].at[0, pl.ds(0, GB)],
                gsems[b],
            ).wait()

    def fire_scatter(c, b):
        seqbase = pl.multiple_of(wid * (BPW // SEQ) + c * SPC, SPC)
        pltpu.async_copy(rows[b], out_hbm.at[pl.ds(seqbase, SPC)], ssems[b])

    def wait_scatter(b):
        pltpu.make_async_copy(
            rows[b], out_hbm.at[pl.ds(0, SPC)], ssems[b]).wait()

    def body(cc, b, fire, stage, skip_scatter_wait=False):
        gb = (b + 2) % NBUF
        if fire:
            if not skip_scatter_wait:
                wait_scatter(gb)
            wait_idx(gb)
            fire_gathers(gb)
        wait_gathers(b)
        if stage:
            stage_idx(cc + NBUF, b)
        fire_scatter(cc, b)

    for c in range(NBUF):
        stage_idx(jnp.int32(c), c)
    wait_idx(0)
    fire_gathers(0)
    wait_idx(1)
    fire_gathers(1)
    body(jnp.int32(0), 0, fire=True, stage=True, skip_scatter_wait=True)
    body(jnp.int32(1), 1, fire=True, stage=True, skip_scatter_wait=True)

    @pl.loop(0, (NCHUNK - 8) // NBUF)
    def _grp(m):
        cc0 = 2 + NBUF * m
        for off in range(NBUF):
            body(cc0 + off, (2 + off) % NBUF, fire=True, stage=True)

    body(jnp.int32(NCHUNK - 6), 2, fire=True, stage=True)
    body(jnp.int32(NCHUNK - 5), 3, fire=True, stage=True)
    body(jnp.int32(NCHUNK - 4), 0, fire=True, stage=False)
    body(jnp.int32(NCHUNK - 3), 1, fire=True, stage=False)
    body(jnp.int32(NCHUNK - 2), 2, fire=False, stage=False)
    body(jnp.int32(NCHUNK - 1), 3, fire=False, stage=False)
    for b in range(NBUF):
        wait_scatter(b)


def kernel(x, table):
    pe = _pe_table()
    x2d = x.reshape(XROWS, GB).astype(jnp.int32)
    g = _sc_lookup(table, x2d)
    g2 = g.reshape(B // 2, 128)
    ot = _transpose_pe(g2, pe)
    return ot.transpose(2, 0, 1)
